# baseline (speedup 1.0000x reference)
import numpy as np
import jax
import jax.numpy as jnp
from jax import lax
from jax.experimental import pallas as pl
from jax.experimental.pallas import tpu as pltpu

ROWS = 128
COLS = 100000
CP = 100352
RA = 16

MIN_P = 0.05
LOG_MIN_P = np.float32(np.log(np.float32(MIN_P)))
TINY = np.float32(np.finfo(np.float32).tiny)
NEG_INF = np.float32(-np.inf)
POS_INF = np.float32(np.inf)

_DCHUNK = 2048


def _threefry_bits(i_u32):
    k0 = np.uint32(0)
    k1 = np.uint32(42)
    ks = (k0, k1, np.uint32(k0 ^ k1 ^ np.uint32(0x1BD11BDA)))
    rot = ((13, 15, 26, 6), (17, 29, 16, 24))
    x0 = jnp.full_like(i_u32, ks[0])
    x1 = i_u32 + ks[1]
    for g in range(5):
        for r in rot[g % 2]:
            x0 = x0 + x1
            x1 = (x1 << np.uint32(r)) | (x1 >> np.uint32(32 - r))
            x1 = x0 ^ x1
        x0 = x0 + ks[(g + 1) % 3]
        x1 = x1 + ks[(g + 2) % 3] + np.uint32(g + 1)
    return x0 ^ x1


def _w_from_bits(bits):
    fb = (bits >> np.uint32(9)) | np.uint32(0x3F800000)
    f = lax.bitcast_convert_type(fb, jnp.float32) - np.float32(1.0)
    u = jnp.maximum(f, TINY)
    return -jnp.log(u)


def _dense_body(l_ref, out_ref):
    i = pl.program_id(0)
    l = l_ref[...]
    rowmax = jnp.max(l, axis=1, keepdims=True)
    thr = rowmax + LOG_MIN_P

    row0 = (i * RA + lax.broadcasted_iota(jnp.int32, (RA, _DCHUNK), 0))
    col_local = lax.broadcasted_iota(jnp.int32, (RA, _DCHUNK), 1)

    def chunk_step(c, carry):
        m, ci = carry
        start = pl.multiple_of(c * _DCHUNK, _DCHUNK)
        lc = l_ref[:, pl.ds(start, _DCHUNK)]
        flat = row0 * COLS + (c * _DCHUNK + col_local)
        w = _w_from_bits(_threefry_bits(flat.astype(jnp.uint32)))
        key = jnp.where(lc >= thr, jnp.exp(thr - lc) * w, POS_INF)
        upd = key < m
        return jnp.where(upd, key, m), jnp.where(upd, c, ci)

    m0 = jnp.full((RA, _DCHUNK), POS_INF, jnp.float32)
    c0 = jnp.zeros((RA, _DCHUNK), jnp.int32)
    m, ci = lax.fori_loop(0, CP // _DCHUNK, chunk_step, (m0, c0))

    mn = jnp.min(m, axis=1, keepdims=True)
    flatcol = ci * _DCHUNK + col_local
    idx = jnp.min(jnp.where(m == mn, flatcol, np.int32(CP)), axis=1)
    out_ref[...] = jnp.broadcast_to(idx[:, None], (RA, 128))


def _dense_sample(logits_p):
    out = pl.pallas_call(
        _dense_body,
        grid=(ROWS // RA,),
        in_specs=[pl.BlockSpec((RA, CP), lambda i: (i, 0))],
        out_specs=pl.BlockSpec((RA, 128), lambda i: (i, 0)),
        out_shape=jax.ShapeDtypeStruct((ROWS, 128), jnp.int32),
    )(logits_p)
    return out[:, :1]


@jax.jit
def kernel(logits):
    logits_p = jnp.pad(logits, ((0, 0), (0, CP - COLS)),
                       constant_values=NEG_INF)
    return _dense_sample(logits_p)


# speedup vs baseline: 1.0273x; 1.0273x over previous
import functools

import numpy as np
import jax
import jax.numpy as jnp
from jax import lax
from jax.experimental import pallas as pl
from jax.experimental.pallas import tpu as pltpu
from jax.experimental.pallas import tpu_sc as plsc

ROWS = 128
COLS = 100000
CP = 100352
NCHUNK = CP // 128
K = 1024
GB = 64
RA = 16

MIN_P = 0.05
LOG_MIN_P = np.float32(np.log(np.float32(MIN_P)))
TINY = np.float32(np.finfo(np.float32).tiny)
NEG_INF = np.float32(-np.inf)


def _threefry_bits(i_u32):
    k0 = np.uint32(0)
    k1 = np.uint32(42)
    ks = (k0, k1, np.uint32(k0 ^ k1 ^ np.uint32(0x1BD11BDA)))
    rot = ((13, 15, 26, 6), (17, 29, 16, 24))
    x0 = jnp.full_like(i_u32, ks[0])
    x1 = i_u32 + ks[1]
    for g in range(5):
        for r in rot[g % 2]:
            x0 = x0 + x1
            x1 = (x1 << np.uint32(r)) | (x1 >> np.uint32(32 - r))
            x1 = x0 ^ x1
        x0 = x0 + ks[(g + 1) % 3]
        x1 = x1 + ks[(g + 2) % 3] + np.uint32(g + 1)
    return x0 ^ x1


def _gumbel_from_bits(bits):
    fb = (bits >> np.uint32(9)) | np.uint32(0x3F800000)
    f = lax.bitcast_convert_type(fb, jnp.float32) - np.float32(1.0)
    u = jnp.maximum(f, TINY)
    return -jnp.log(-jnp.log(u))


_DCHUNK = 2048


def _dense_body(l_ref, out_ref, scores_ref):
    i = pl.program_id(0)
    l = l_ref[...]
    thr = jnp.max(l, axis=1, keepdims=True) + LOG_MIN_P

    row0 = (i * RA + lax.broadcasted_iota(jnp.int32, (RA, _DCHUNK), 0))
    col_local = lax.broadcasted_iota(jnp.int32, (RA, _DCHUNK), 1)

    def chunk_step(c, carry):
        start = pl.multiple_of(c * _DCHUNK, _DCHUNK)
        lc = l_ref[:, pl.ds(start, _DCHUNK)]
        flat = row0 * COLS + (c * _DCHUNK + col_local)
        g = _gumbel_from_bits(_threefry_bits(flat.astype(jnp.uint32)))
        scores_ref[:, pl.ds(start, _DCHUNK)] = jnp.where(
            lc >= thr, lc + g, NEG_INF)
        return carry

    lax.fori_loop(0, CP // _DCHUNK, chunk_step, 0)

    s = scores_ref[...]
    mx = jnp.max(s, axis=1, keepdims=True)
    col = lax.broadcasted_iota(jnp.int32, (RA, CP), 1)
    idx = jnp.min(jnp.where(s == mx, col, np.int32(CP)), axis=1)
    out_ref[...] = jnp.broadcast_to(idx[:, None], (RA, 128))


def _dense_sample(logits_p):
    out = pl.pallas_call(
        _dense_body,
        grid=(ROWS // RA,),
        in_specs=[pl.BlockSpec((RA, CP), lambda i: (i, 0))],
        out_specs=pl.BlockSpec((RA, 128), lambda i: (i, 0)),
        out_shape=jax.ShapeDtypeStruct((ROWS, 128), jnp.int32),
        scratch_shapes=[pltpu.VMEM((RA, CP), jnp.float32)],
    )(logits_p)
    return out[:, :1]


def _stats_body(l3_ref, cmax_ref, thr_ref):
    l3 = l3_ref[...]
    cm = jnp.max(l3, axis=2)
    cmax_ref[...] = cm
    m = jnp.max(cm, axis=1, keepdims=True)
    thr_ref[...] = jnp.broadcast_to(m + LOG_MIN_P, (RA, 16))


def _stats(logits3):
    return pl.pallas_call(
        _stats_body,
        grid=(ROWS // RA,),
        in_specs=[pl.BlockSpec((RA, NCHUNK, 128), lambda i: (i, 0, 0))],
        out_specs=[
            pl.BlockSpec((RA, NCHUNK), lambda i: (i, 0)),
            pl.BlockSpec((RA, 16), lambda i: (i, 0)),
        ],
        out_shape=[
            jax.ShapeDtypeStruct((ROWS, NCHUNK), jnp.float32),
            jax.ShapeDtypeStruct((ROWS, 16), jnp.float32),
        ],
    )(logits3)


_ROWS_PER_TILE = ROWS // 32


def _sc_compact(table, cmax, thr):
    mesh = plsc.VectorSubcoreMesh(
        core_axis_name="c", subcore_axis_name="s", num_cores=2,
        num_subcores=16)

    @functools.partial(
        pl.kernel,
        compiler_params=pltpu.CompilerParams(needs_layout_passes=False),
        out_type=(
            jax.ShapeDtypeStruct((ROWS, K), jnp.int32),
            jax.ShapeDtypeStruct((ROWS, K), jnp.float32),
            jax.ShapeDtypeStruct((ROWS, 16), jnp.int32),
        ),
        mesh=mesh,
        scratch_types=(
            pltpu.VMEM((NCHUNK,), jnp.float32),
            pltpu.VMEM((16,), jnp.float32),
            pltpu.VMEM((NCHUNK + 2 * GB,), jnp.int32),
            pltpu.VMEM((GB,), jnp.int32),
            pltpu.VMEM((GB, 128), jnp.float32),
            pltpu.VMEM((K + 16,), jnp.int32),
            pltpu.VMEM((K + 16,), jnp.float32),
            pltpu.VMEM((16,), jnp.int32),
            pltpu.SemaphoreType.DMA,
        ),
    )
    def body(table_hbm, cmax_hbm, thr_hbm, cidx_hbm, cval_hbm, cnt_hbm,
             cm_buf, thr_buf, ids_buf, idxv_buf, gbuf, cidx_buf, cval_buf,
             cnt_buf, sem):
        wid = lax.axis_index("s") * 2 + lax.axis_index("c")
        iota16 = lax.iota(jnp.int32, 16)

        def row_body(ri, _):
            r = wid * _ROWS_PER_TILE + ri
            rbase = r * NCHUNK
            pltpu.sync_copy(cmax_hbm.at[r], cm_buf)
            pltpu.sync_copy(thr_hbm.at[r], thr_buf)
            thrv = thr_buf[...]

            def fill(j, _):
                cidx_buf[pl.ds(j * 16, 16)] = jnp.full((16,), COLS, jnp.int32)
                cval_buf[pl.ds(j * 16, 16)] = jnp.full((16,), NEG_INF,
                                                       jnp.float32)
                return 0
            lax.fori_loop(0, K // 16, fill, 0)

            def fscan(j, nc):
                v = cm_buf[pl.ds(j * 16, 16)]
                mask = v >= thrv
                pc = jnp.sum(mask.astype(jnp.int32))

                @pl.when(pc > 0)
                def _():
                    ids = rbase + j * 16 + iota16
                    plsc.store_compressed(ids_buf.at[pl.ds(nc, 16)], ids,
                                          mask=mask)
                return nc + pc
            nc = lax.fori_loop(0, NCHUNK // 16, fscan, jnp.int32(0))

            def pad(k2, _):
                ids_buf[pl.ds(nc + k2 * 16, 16)] = jnp.full((16,), rbase,
                                                            jnp.int32)
                return 0
            lax.fori_loop(0, GB // 16, pad, 0)

            nbatch = (nc + GB - 1) // GB

            def batch_body(b, ncand):
                def cp(k2, _):
                    idxv_buf[pl.ds(k2 * 16, 16)] = (
                        ids_buf[pl.ds(b * GB + k2 * 16, 16)])
                    return 0
                lax.fori_loop(0, GB // 16, cp, 0)
                pltpu.async_copy(table_hbm.at[idxv_buf], gbuf, sem).wait()
                nb = jnp.minimum(nc - b * GB, GB)

                def grp_body(k2, ncand):
                    chunkvec = k2 * 16 + iota16
                    idsv = idxv_buf[pl.ds(k2 * 16, 16)]
                    colb = (idsv - rbase) * 128
                    valid = chunkvec < nb

                    def vloop(j, ncand):
                        v = plsc.load_gather(
                            gbuf, [chunkvec, jnp.full((16,), j, jnp.int32)])
                        mask = jnp.logical_and(v >= thrv, valid)
                        pc = jnp.sum(mask.astype(jnp.int32))

                        @pl.when(pc > 0)
                        def _():
                            off = jnp.minimum(ncand, K)
                            cols = colb + j
                            plsc.store_compressed(
                                cidx_buf.at[pl.ds(off, 16)], cols, mask=mask)
                            plsc.store_compressed(
                                cval_buf.at[pl.ds(off, 16)], v, mask=mask)
                        return ncand + pc
                    return lax.fori_loop(0, 128, vloop, ncand)
                return lax.fori_loop(0, GB // 16, grp_body, ncand)
            ncand = lax.fori_loop(0, nbatch, batch_body, jnp.int32(0))

            pltpu.sync_copy(cidx_buf.at[pl.ds(0, K)], cidx_hbm.at[r])
            pltpu.sync_copy(cval_buf.at[pl.ds(0, K)], cval_hbm.at[r])
            cnt_buf[...] = jnp.full((16,), ncand, jnp.int32)
            pltpu.sync_copy(cnt_buf, cnt_hbm.at[r])
            return 0

        lax.fori_loop(0, _ROWS_PER_TILE, row_body, 0)

    return body(table, cmax, thr)


def _pick_body(ci_ref, cv_ref, out_ref):
    idx = ci_ref[...]
    val = cv_ref[...]
    row = lax.broadcasted_iota(jnp.int32, (ROWS, K), 0)
    flat = row * COLS + idx
    g = _gumbel_from_bits(_threefry_bits(flat.astype(jnp.uint32)))
    score = val + g
    mx = jnp.max(score, axis=1, keepdims=True)
    win = jnp.min(jnp.where(score == mx, idx, np.int32(COLS + 1)), axis=1)
    out_ref[...] = jnp.broadcast_to(win[:, None], (ROWS, 128))


def _pick(ci, cv):
    out = pl.pallas_call(
        _pick_body,
        in_specs=[
            pl.BlockSpec((ROWS, K), lambda: (0, 0)),
            pl.BlockSpec((ROWS, K), lambda: (0, 0)),
        ],
        out_specs=pl.BlockSpec((ROWS, 128), lambda: (0, 0)),
        out_shape=jax.ShapeDtypeStruct((ROWS, 128), jnp.int32),
    )(ci, cv)
    return out[:, :1]


@jax.jit
def kernel(logits):
    logits_p = jnp.pad(logits, ((0, 0), (0, CP - COLS)),
                       constant_values=NEG_INF)
    cmax, thr = _stats(logits_p.reshape(ROWS, NCHUNK, 128))
    ci, cv, cnt = _sc_compact(logits_p.reshape(ROWS * NCHUNK, 128),
                              cmax, thr)
    overflow = jnp.any(cnt[:, 0] > K)
    return lax.cond(overflow,
                    lambda: _dense_sample(logits_p),
                    lambda: _pick(ci, cv))


# speedup vs baseline: 1.2870x; 1.2527x over previous
import functools

import numpy as np
import jax
import jax.numpy as jnp
from jax import lax
from jax.experimental import pallas as pl
from jax.experimental.pallas import tpu as pltpu
from jax.experimental.pallas import tpu_sc as plsc

ROWS = 128
COLS = 100000
CP = 100352
NCHUNK = CP // 128
K = 1024
GB = 128
RA = 16

MIN_P = 0.05
LOG_MIN_P = np.float32(np.log(np.float32(MIN_P)))
TINY = np.float32(np.finfo(np.float32).tiny)
NEG_INF = np.float32(-np.inf)


def _threefry_bits(i_u32):
    k0 = np.uint32(0)
    k1 = np.uint32(42)
    ks = (k0, k1, np.uint32(k0 ^ k1 ^ np.uint32(0x1BD11BDA)))
    rot = ((13, 15, 26, 6), (17, 29, 16, 24))
    x0 = jnp.full_like(i_u32, ks[0])
    x1 = i_u32 + ks[1]
    for g in range(5):
        for r in rot[g % 2]:
            x0 = x0 + x1
            x1 = (x1 << np.uint32(r)) | (x1 >> np.uint32(32 - r))
            x1 = x0 ^ x1
        x0 = x0 + ks[(g + 1) % 3]
        x1 = x1 + ks[(g + 2) % 3] + np.uint32(g + 1)
    return x0 ^ x1


def _gumbel_from_bits(bits):
    fb = (bits >> np.uint32(9)) | np.uint32(0x3F800000)
    f = lax.bitcast_convert_type(fb, jnp.float32) - np.float32(1.0)
    u = jnp.maximum(f, TINY)
    return -jnp.log(-jnp.log(u))


_DCHUNK = 2048


def _dense_body(l_ref, out_ref, scores_ref):
    i = pl.program_id(0)
    l = l_ref[...]
    thr = jnp.max(l, axis=1, keepdims=True) + LOG_MIN_P

    row0 = (i * RA + lax.broadcasted_iota(jnp.int32, (RA, _DCHUNK), 0))
    col_local = lax.broadcasted_iota(jnp.int32, (RA, _DCHUNK), 1)

    def chunk_step(c, carry):
        start = pl.multiple_of(c * _DCHUNK, _DCHUNK)
        lc = l_ref[:, pl.ds(start, _DCHUNK)]
        flat = row0 * COLS + (c * _DCHUNK + col_local)
        g = _gumbel_from_bits(_threefry_bits(flat.astype(jnp.uint32)))
        scores_ref[:, pl.ds(start, _DCHUNK)] = jnp.where(
            lc >= thr, lc + g, NEG_INF)
        return carry

    lax.fori_loop(0, CP // _DCHUNK, chunk_step, 0)

    s = scores_ref[...]
    mx = jnp.max(s, axis=1, keepdims=True)
    col = lax.broadcasted_iota(jnp.int32, (RA, CP), 1)
    idx = jnp.min(jnp.where(s == mx, col, np.int32(CP)), axis=1)
    out_ref[...] = jnp.broadcast_to(idx[:, None], (RA, 128))


def _dense_sample(logits_p):
    out = pl.pallas_call(
        _dense_body,
        grid=(ROWS // RA,),
        in_specs=[pl.BlockSpec((RA, CP), lambda i: (i, 0))],
        out_specs=pl.BlockSpec((RA, 128), lambda i: (i, 0)),
        out_shape=jax.ShapeDtypeStruct((ROWS, 128), jnp.int32),
        scratch_shapes=[pltpu.VMEM((RA, CP), jnp.float32)],
    )(logits_p)
    return out[:, :1]


def _stats_body(l3_ref, cmax_ref, thr_ref):
    l3 = l3_ref[...]
    cm = jnp.max(l3, axis=2)
    cmax_ref[...] = cm
    m = jnp.max(cm, axis=1, keepdims=True)
    thr_ref[...] = jnp.broadcast_to(m + LOG_MIN_P, (RA, 16))


def _stats(logits3):
    return pl.pallas_call(
        _stats_body,
        grid=(ROWS // RA,),
        in_specs=[pl.BlockSpec((RA, NCHUNK, 128), lambda i: (i, 0, 0))],
        out_specs=[
            pl.BlockSpec((RA, NCHUNK), lambda i: (i, 0)),
            pl.BlockSpec((RA, 16), lambda i: (i, 0)),
        ],
        out_shape=[
            jax.ShapeDtypeStruct((ROWS, NCHUNK), jnp.float32),
            jax.ShapeDtypeStruct((ROWS, 16), jnp.float32),
        ],
    )(logits3)


_RPT = ROWS // 32


def _sc_compact(table, cmax, thr):
    mesh = plsc.VectorSubcoreMesh(
        core_axis_name="c", subcore_axis_name="s", num_cores=2,
        num_subcores=16)

    @functools.partial(
        pl.kernel,
        compiler_params=pltpu.CompilerParams(needs_layout_passes=False),
        out_type=(
            jax.ShapeDtypeStruct((ROWS, K), jnp.int32),
            jax.ShapeDtypeStruct((ROWS, K), jnp.float32),
            jax.ShapeDtypeStruct((ROWS, 16), jnp.int32),
        ),
        mesh=mesh,
        scratch_types=(
            pltpu.VMEM((_RPT, NCHUNK), jnp.float32),
            pltpu.VMEM((_RPT, 16), jnp.float32),
            pltpu.VMEM((NCHUNK + 2 * GB,), jnp.int32),
            pltpu.VMEM((GB,), jnp.int32),
            pltpu.VMEM((GB, 128), jnp.float32),
            pltpu.VMEM((K + 16,), jnp.int32),
            pltpu.VMEM((K + 16,), jnp.float32),
            pltpu.VMEM((16,), jnp.int32),
            pltpu.SemaphoreType.DMA,
        ),
    )
    def body(table_hbm, cmax_hbm, thr_hbm, cidx_hbm, cval_hbm, cnt_hbm,
             cm_buf, thr_buf, ids_buf, idxv_buf, gbuf, cidx_buf, cval_buf,
             cnt_buf, sem):
        wid = lax.axis_index("s") * 2 + lax.axis_index("c")
        iota16 = lax.iota(jnp.int32, 16)

        pltpu.sync_copy(cmax_hbm.at[pl.ds(wid * _RPT, _RPT)], cm_buf)
        pltpu.sync_copy(thr_hbm.at[pl.ds(wid * _RPT, _RPT)], thr_buf)

        def row_body(ri, _):
            r = wid * _RPT + ri
            rbase = r * NCHUNK
            thrv = thr_buf[ri]

            def fscan(j, nc):
                v = cm_buf[ri, pl.ds(j * 16, 16)]
                mask = v >= thrv
                pc = plsc.all_reduce_population_count(mask)[0]
                ids = rbase + j * 16 + iota16
                plsc.store_compressed(ids_buf.at[pl.ds(nc, 16)], ids,
                                      mask=mask)
                return nc + pc
            nc = lax.fori_loop(0, NCHUNK // 16, fscan, jnp.int32(0))

            def pad(k2, _):
                ids_buf[pl.ds(nc + k2 * 16, 16)] = jnp.full((16,), rbase,
                                                            jnp.int32)
                return 0
            lax.fori_loop(0, GB // 16, pad, 0)

            nbatch = (nc + GB - 1) // GB

            def batch_body(b, ncand):
                def cp(k2, _):
                    idxv_buf[pl.ds(k2 * 16, 16)] = (
                        ids_buf[pl.ds(b * GB + k2 * 16, 16)])
                    return 0
                lax.fori_loop(0, GB // 16, cp, 0)
                pltpu.async_copy(table_hbm.at[idxv_buf], gbuf, sem).wait()
                nb = jnp.minimum(nc - b * GB, GB)

                def grp_body(k2, ncand):
                    chunkvec = k2 * 16 + iota16
                    idsv = idxv_buf[pl.ds(k2 * 16, 16)]
                    colb = (idsv - rbase) * 128
                    valid = chunkvec < nb

                    def vloop(j, ncand):
                        v = plsc.load_gather(
                            gbuf, [chunkvec, jnp.full((16,), j, jnp.int32)])
                        mask = jnp.logical_and(v >= thrv, valid)
                        pc = plsc.all_reduce_population_count(mask)[0]
                        off = jnp.minimum(ncand, K)
                        cols = colb + j
                        plsc.store_compressed(
                            cidx_buf.at[pl.ds(off, 16)], cols, mask=mask)
                        plsc.store_compressed(
                            cval_buf.at[pl.ds(off, 16)], v, mask=mask)
                        return ncand + pc
                    return lax.fori_loop(0, 128, vloop, ncand)
                ngrp = (nb + 15) // 16
                return lax.fori_loop(0, ngrp, grp_body, ncand)
            ncand = lax.fori_loop(0, nbatch, batch_body, jnp.int32(0))

            pltpu.sync_copy(cidx_buf.at[pl.ds(0, K)], cidx_hbm.at[r])
            pltpu.sync_copy(cval_buf.at[pl.ds(0, K)], cval_hbm.at[r])
            cnt_buf[...] = jnp.full((16,), ncand, jnp.int32)
            pltpu.sync_copy(cnt_buf, cnt_hbm.at[r])
            return 0

        lax.fori_loop(0, _RPT, row_body, 0)

    return body(table, cmax, thr)


def _pick_body(ci_ref, cv_ref, cnt_ref, out_ref):
    idx = ci_ref[...]
    val = cv_ref[...]
    cnt = cnt_ref[...][:, :1]
    pos = lax.broadcasted_iota(jnp.int32, (ROWS, K), 1)
    live = pos < cnt
    row = lax.broadcasted_iota(jnp.int32, (ROWS, K), 0)
    flat = row * COLS + jnp.where(live, idx, 0)
    g = _gumbel_from_bits(_threefry_bits(flat.astype(jnp.uint32)))
    score = jnp.where(live, val + g, NEG_INF)
    mx = jnp.max(score, axis=1, keepdims=True)
    win = jnp.min(jnp.where(score == mx, idx, np.int32(COLS + 1)), axis=1)
    out_ref[...] = jnp.broadcast_to(win[:, None], (ROWS, 128))


def _pick(ci, cv, cnt):
    out = pl.pallas_call(
        _pick_body,
        in_specs=[
            pl.BlockSpec((ROWS, K), lambda: (0, 0)),
            pl.BlockSpec((ROWS, K), lambda: (0, 0)),
            pl.BlockSpec((ROWS, 16), lambda: (0, 0)),
        ],
        out_specs=pl.BlockSpec((ROWS, 128), lambda: (0, 0)),
        out_shape=jax.ShapeDtypeStruct((ROWS, 128), jnp.int32),
    )(ci, cv, cnt)
    return out[:, :1]


@jax.jit
def kernel(logits):
    logits_p = jnp.pad(logits, ((0, 0), (0, CP - COLS)),
                       constant_values=NEG_INF)
    cmax, thr = _stats(logits_p.reshape(ROWS, NCHUNK, 128))
    ci, cv, cnt = _sc_compact(logits_p.reshape(ROWS * NCHUNK, 128),
                              cmax, thr)
    overflow = jnp.any(cnt[:, 0] > K)
    return lax.cond(overflow,
                    lambda: _dense_sample(logits_p),
                    lambda: _pick(ci, cv, cnt))


# speedup vs baseline: 1.6559x; 1.2867x over previous
import functools

import numpy as np
import jax
import jax.numpy as jnp
from jax import lax
from jax.experimental import pallas as pl
from jax.experimental.pallas import tpu as pltpu
from jax.experimental.pallas import tpu_sc as plsc

ROWS = 128
COLS = 100000
CP = 100352
NCHUNK = CP // 128
K = 1024
GB = 128
RA = 16

MIN_P = 0.05
LOG_MIN_P = np.float32(np.log(np.float32(MIN_P)))
TINY = np.float32(np.finfo(np.float32).tiny)
NEG_INF = np.float32(-np.inf)


def _threefry_bits(i_u32):
    k0 = np.uint32(0)
    k1 = np.uint32(42)
    ks = (k0, k1, np.uint32(k0 ^ k1 ^ np.uint32(0x1BD11BDA)))
    rot = ((13, 15, 26, 6), (17, 29, 16, 24))
    x0 = jnp.full_like(i_u32, ks[0])
    x1 = i_u32 + ks[1]
    for g in range(5):
        for r in rot[g % 2]:
            x0 = x0 + x1
            x1 = (x1 << np.uint32(r)) | (x1 >> np.uint32(32 - r))
            x1 = x0 ^ x1
        x0 = x0 + ks[(g + 1) % 3]
        x1 = x1 + ks[(g + 2) % 3] + np.uint32(g + 1)
    return x0 ^ x1


def _gumbel_from_bits(bits):
    fb = (bits >> np.uint32(9)) | np.uint32(0x3F800000)
    f = lax.bitcast_convert_type(fb, jnp.float32) - np.float32(1.0)
    u = jnp.maximum(f, TINY)
    return -jnp.log(-jnp.log(u))


_DCHUNK = 2048


def _dense_body(l_ref, out_ref, scores_ref):
    i = pl.program_id(0)
    l = l_ref[...]
    thr = jnp.max(l, axis=1, keepdims=True) + LOG_MIN_P

    row0 = (i * RA + lax.broadcasted_iota(jnp.int32, (RA, _DCHUNK), 0))
    col_local = lax.broadcasted_iota(jnp.int32, (RA, _DCHUNK), 1)

    def chunk_step(c, carry):
        start = pl.multiple_of(c * _DCHUNK, _DCHUNK)
        lc = l_ref[:, pl.ds(start, _DCHUNK)]
        flat = row0 * COLS + (c * _DCHUNK + col_local)
        g = _gumbel_from_bits(_threefry_bits(flat.astype(jnp.uint32)))
        scores_ref[:, pl.ds(start, _DCHUNK)] = jnp.where(
            lc >= thr, lc + g, NEG_INF)
        return carry

    lax.fori_loop(0, CP // _DCHUNK, chunk_step, 0)

    s = scores_ref[...]
    mx = jnp.max(s, axis=1, keepdims=True)
    col = lax.broadcasted_iota(jnp.int32, (RA, CP), 1)
    idx = jnp.min(jnp.where(s == mx, col, np.int32(CP)), axis=1)
    out_ref[...] = jnp.broadcast_to(idx[:, None], (RA, 128))


def _dense_sample(logits_p):
    out = pl.pallas_call(
        _dense_body,
        grid=(ROWS // RA,),
        in_specs=[pl.BlockSpec((RA, CP), lambda i: (i, 0))],
        out_specs=pl.BlockSpec((RA, 128), lambda i: (i, 0)),
        out_shape=jax.ShapeDtypeStruct((ROWS, 128), jnp.int32),
        scratch_shapes=[pltpu.VMEM((RA, CP), jnp.float32)],
    )(logits_p)
    return out[:, :1]


def _stats_body(l_ref, cmax_ref, thr_ref, tbl_ref):
    pieces = []
    for c in range(48):
        l2 = l_ref[:, pl.ds(c * 2048, 2048)]
        tbl_ref[:, pl.ds(c * 2048, 2048)] = l2
        pieces.append(jnp.max(l2.reshape(RA, 16, 128), axis=2))
    tail = []
    for t in range(13):
        lt = l_ref[:, pl.ds(98304 + t * 128, 128)]
        tbl_ref[:, pl.ds(98304 + t * 128, 128)] = lt
        tail.append(jnp.max(lt, axis=1))
    l32 = l_ref[:, pl.ds(99968, 32)]
    tbl_ref[:, pl.ds(99968, 384)] = jnp.concatenate(
        [l32, jnp.full((RA, 352), NEG_INF, jnp.float32)], axis=1)
    tail.append(jnp.max(l32, axis=1))
    tail.append(jnp.full((RA,), NEG_INF, jnp.float32))
    tail.append(jnp.full((RA,), NEG_INF, jnp.float32))
    cm = jnp.concatenate(pieces + [jnp.stack(tail, axis=1)], axis=1)
    cmax_ref[...] = cm
    m = jnp.max(cm, axis=1, keepdims=True)
    thr_ref[...] = jnp.broadcast_to(m + LOG_MIN_P, (RA, 16))


def _stats(logits):
    return pl.pallas_call(
        _stats_body,
        grid=(ROWS // RA,),
        in_specs=[pl.BlockSpec((RA, COLS), lambda i: (i, 0))],
        out_specs=[
            pl.BlockSpec((RA, NCHUNK), lambda i: (i, 0)),
            pl.BlockSpec((RA, 16), lambda i: (i, 0)),
            pl.BlockSpec((RA, CP), lambda i: (i, 0)),
        ],
        out_shape=[
            jax.ShapeDtypeStruct((ROWS, NCHUNK), jnp.float32),
            jax.ShapeDtypeStruct((ROWS, 16), jnp.float32),
            jax.ShapeDtypeStruct((ROWS, CP), jnp.float32),
        ],
    )(logits)


_RPT = ROWS // 32


def _sc_compact(table, cmax, thr):
    mesh = plsc.VectorSubcoreMesh(
        core_axis_name="c", subcore_axis_name="s", num_cores=2,
        num_subcores=16)

    @functools.partial(
        pl.kernel,
        compiler_params=pltpu.CompilerParams(needs_layout_passes=False),
        out_type=(
            jax.ShapeDtypeStruct((ROWS, K), jnp.int32),
            jax.ShapeDtypeStruct((ROWS, K), jnp.float32),
            jax.ShapeDtypeStruct((ROWS, 16), jnp.int32),
        ),
        mesh=mesh,
        scratch_types=(
            pltpu.VMEM((_RPT, NCHUNK), jnp.float32),
            pltpu.VMEM((_RPT, 16), jnp.float32),
            pltpu.VMEM((NCHUNK + 2 * GB,), jnp.int32),
            pltpu.VMEM((GB,), jnp.int32),
            pltpu.VMEM((GB, 128), jnp.float32),
            pltpu.VMEM((K + 16,), jnp.int32),
            pltpu.VMEM((K + 16,), jnp.float32),
            pltpu.VMEM((16,), jnp.int32),
            pltpu.SemaphoreType.DMA,
        ),
    )
    def body(table_hbm, cmax_hbm, thr_hbm, cidx_hbm, cval_hbm, cnt_hbm,
             cm_buf, thr_buf, ids_buf, idxv_buf, gbuf, cidx_buf, cval_buf,
             cnt_buf, sem):
        wid = lax.axis_index("s") * 2 + lax.axis_index("c")
        iota16 = lax.iota(jnp.int32, 16)

        pltpu.sync_copy(cmax_hbm.at[pl.ds(wid * _RPT, _RPT)], cm_buf)
        pltpu.sync_copy(thr_hbm.at[pl.ds(wid * _RPT, _RPT)], thr_buf)

        def row_body(ri, _):
            r = wid * _RPT + ri
            rbase = r * NCHUNK
            thrv = thr_buf[ri]

            def fscan(j, nc):
                v = cm_buf[ri, pl.ds(j * 16, 16)]
                mask = v >= thrv
                pc = plsc.all_reduce_population_count(mask)[0]
                ids = rbase + j * 16 + iota16
                plsc.store_compressed(ids_buf.at[pl.ds(nc, 16)], ids,
                                      mask=mask)
                return nc + pc
            nc = lax.fori_loop(0, NCHUNK // 16, fscan, jnp.int32(0))

            def pad(k2, _):
                ids_buf[pl.ds(nc + k2 * 16, 16)] = jnp.full((16,), rbase,
                                                            jnp.int32)
                return 0
            lax.fori_loop(0, GB // 16, pad, 0)

            nbatch = (nc + GB - 1) // GB

            def batch_body(b, ncand):
                def cp(k2, _):
                    idxv_buf[pl.ds(k2 * 16, 16)] = (
                        ids_buf[pl.ds(b * GB + k2 * 16, 16)])
                    return 0
                lax.fori_loop(0, GB // 16, cp, 0)
                pltpu.async_copy(table_hbm.at[idxv_buf], gbuf, sem).wait()
                nb = jnp.minimum(nc - b * GB, GB)

                def grp_body(k2, ncand):
                    chunkvec = k2 * 16 + iota16
                    idsv = idxv_buf[pl.ds(k2 * 16, 16)]
                    colb = (idsv - rbase) * 128
                    valid = chunkvec < nb

                    def vloop(j, ncand):
                        v = plsc.load_gather(
                            gbuf, [chunkvec, jnp.full((16,), j, jnp.int32)])
                        mask = jnp.logical_and(v >= thrv, valid)
                        pc = plsc.all_reduce_population_count(mask)[0]
                        off = jnp.minimum(ncand, K)
                        cols = colb + j
                        plsc.store_compressed(
                            cidx_buf.at[pl.ds(off, 16)], cols, mask=mask)
                        plsc.store_compressed(
                            cval_buf.at[pl.ds(off, 16)], v, mask=mask)
                        return ncand + pc
                    return lax.fori_loop(0, 128, vloop, ncand)
                ngrp = (nb + 15) // 16
                return lax.fori_loop(0, ngrp, grp_body, ncand)
            ncand = lax.fori_loop(0, nbatch, batch_body, jnp.int32(0))

            pltpu.sync_copy(cidx_buf.at[pl.ds(0, K)], cidx_hbm.at[r])
            pltpu.sync_copy(cval_buf.at[pl.ds(0, K)], cval_hbm.at[r])
            cnt_buf[...] = jnp.full((16,), ncand, jnp.int32)
            pltpu.sync_copy(cnt_buf, cnt_hbm.at[r])
            return 0

        lax.fori_loop(0, _RPT, row_body, 0)

    return body(table, cmax, thr)


def _pick_body(ci_ref, cv_ref, cnt_ref, out_ref):
    idx = ci_ref[...]
    val = cv_ref[...]
    cnt = cnt_ref[...][:, :1]
    pos = lax.broadcasted_iota(jnp.int32, (ROWS, K), 1)
    live = pos < cnt
    row = lax.broadcasted_iota(jnp.int32, (ROWS, K), 0)
    flat = row * COLS + jnp.where(live, idx, 0)
    g = _gumbel_from_bits(_threefry_bits(flat.astype(jnp.uint32)))
    score = jnp.where(live, val + g, NEG_INF)
    mx = jnp.max(score, axis=1, keepdims=True)
    win = jnp.min(jnp.where(score == mx, idx, np.int32(COLS + 1)), axis=1)
    out_ref[...] = jnp.broadcast_to(win[:, None], (ROWS, 128))


def _pick(ci, cv, cnt):
    out = pl.pallas_call(
        _pick_body,
        in_specs=[
            pl.BlockSpec((ROWS, K), lambda: (0, 0)),
            pl.BlockSpec((ROWS, K), lambda: (0, 0)),
            pl.BlockSpec((ROWS, 16), lambda: (0, 0)),
        ],
        out_specs=pl.BlockSpec((ROWS, 128), lambda: (0, 0)),
        out_shape=jax.ShapeDtypeStruct((ROWS, 128), jnp.int32),
    )(ci, cv, cnt)
    return out[:, :1]


@jax.jit
def kernel(logits):
    cmax, thr, tbl = _stats(logits)
    ci, cv, cnt = _sc_compact(tbl.reshape(ROWS * NCHUNK, 128), cmax, thr)
    overflow = jnp.any(cnt[:, 0] > K)
    return lax.cond(overflow,
                    lambda: _dense_sample(tbl),
                    lambda: _pick(ci, cv, cnt))
